# baseline (speedup 1.0000x reference)
import functools

import jax
import jax.numpy as jnp
from jax import lax
from jax.experimental import pallas as pl
from jax.experimental.pallas import tpu as pltpu
from jax.experimental.pallas import tpu_sc as plsc


def _vq_argmin_kernel(z_ref, w_ref, idx_ref, loss_ref, cl_ref, el_ref):
    zb = z_ref[0]
    W = w_ref[...]
    D, N = zb.shape
    K = W.shape[1]

    zt = zb.T
    wt = W.T

    S2 = jax.lax.dot_general(zt * jnp.float32(-2.0), W,
                             (((1,), (0,)), ((), ())),
                             preferred_element_type=jnp.float32)
    zsq = jnp.sum(zt * zt, axis=1)
    wsq = jnp.sum(wt * wt, axis=1)
    d = (zsq[:, None] + wsq[None, :]) + S2

    m = jnp.min(d, axis=1)
    iota_row = jax.lax.broadcasted_iota(jnp.int32, (1, K), 1).astype(
        jnp.float32)
    closest = jnp.min(jnp.where(d == m[:, None], iota_row,
                                jnp.float32(K)), axis=1)

    idx_ref[0, 0] = closest.astype(jnp.int32)

    c = m * jnp.float32(1.0 / D)
    cl_ref[0, 0] = c
    el_ref[0, 0] = c
    loss_ref[0, 0] = c * jnp.float32(0.25) + c


def _tc_argmin(z3, W):
    B, D, N = z3.shape
    K = W.shape[1]
    f32 = jnp.float32
    return pl.pallas_call(
        _vq_argmin_kernel,
        grid=(B,),
        in_specs=[
            pl.BlockSpec((1, D, N), lambda b: (b, 0, 0)),
            pl.BlockSpec((D, K), lambda b: (0, 0)),
        ],
        out_specs=[
            pl.BlockSpec((1, 1, N), lambda b: (b, 0, 0)),
            pl.BlockSpec((1, 1, N), lambda b: (b, 0, 0)),
            pl.BlockSpec((1, 1, N), lambda b: (b, 0, 0)),
            pl.BlockSpec((1, 1, N), lambda b: (b, 0, 0)),
        ],
        out_shape=[
            jax.ShapeDtypeStruct((B, 1, N), jnp.int32),
            jax.ShapeDtypeStruct((B, 1, N), f32),
            jax.ShapeDtypeStruct((B, 1, N), f32),
            jax.ShapeDtypeStruct((B, 1, N), f32),
        ],
        compiler_params=pltpu.CompilerParams(
            dimension_semantics=("parallel",)),
    )(z3, W)


def _sc_gather(w_flat, idx_flat, B, D, N, K):
    info = plsc.get_sparse_core_info()
    NC, NS, L = info.num_cores, info.num_subcores, info.num_lanes
    NW = NC * NS
    WPB = NW // B
    CH = D // WPB
    U = 4

    mesh = plsc.VectorSubcoreMesh(core_axis_name="c", subcore_axis_name="s")

    @functools.partial(
        pl.kernel, mesh=mesh,
        out_type=jax.ShapeDtypeStruct((B * D * N,), jnp.float32),
        scratch_types=[
            pltpu.VMEM((CH * K,), jnp.float32),
            pltpu.VMEM((N,), jnp.int32),
            pltpu.VMEM((CH * N,), jnp.float32),
        ],
        compiler_params=pltpu.CompilerParams(needs_layout_passes=False),
    )
    def k(w_hbm, idx_hbm, out_hbm, wv, idxv, outv):
        wid = lax.axis_index("s") * NC + lax.axis_index("c")
        b = wid // WPB
        d0 = (wid % WPB) * CH
        pltpu.sync_copy(idx_hbm.at[pl.ds(b * N, N)], idxv)
        pltpu.sync_copy(w_hbm.at[pl.ds(d0 * K, CH * K)], wv)
        for dd in range(CH):
            base_w = dd * K
            base_o = dd * N

            def body(j, carry, base_w=base_w, base_o=base_o):
                for u in range(U):
                    off = (j * U + u) * L
                    iv = idxv[pl.ds(off, L)]
                    outv[pl.ds(base_o + off, L)] = plsc.load_gather(
                        wv, [iv + base_w])
                return carry

            lax.fori_loop(0, N // L // U, body, 0)
        pltpu.sync_copy(outv, out_hbm.at[pl.ds((b * D + d0) * N, CH * N)])

    return k(w_flat, idx_flat)


def kernel(z, W):
    B, D, H, Wd = z.shape
    N = H * Wd
    K = W.shape[1]
    z3 = z.reshape(B, D, N)
    closest, loss, cl, el = _tc_argmin(z3, W)
    qflat = _sc_gather(W.reshape(D * K), closest.reshape(B * N), B, D, N, K)
    shp = (B, H, Wd)
    return (qflat.reshape(z.shape), loss.reshape(shp), cl.reshape(shp),
            el.reshape(shp))


# speedup vs baseline: 1.2289x; 1.2289x over previous
import functools

import jax
import jax.numpy as jnp
from jax import lax
from jax.experimental import pallas as pl
from jax.experimental.pallas import tpu as pltpu
from jax.experimental.pallas import tpu_sc as plsc


def _vq_argmin_kernel(z_ref, w_ref, idx_ref, loss_ref, cl_ref, el_ref):
    zb = z_ref[0]
    W = w_ref[...]
    D, N = zb.shape
    K = W.shape[1]

    wt = W.T

    S2 = jax.lax.dot_general(wt * jnp.float32(-2.0), zb,
                             (((1,), (0,)), ((), ())),
                             preferred_element_type=jnp.float32)
    zsq = jnp.sum(zb * zb, axis=0, keepdims=True)
    wsq = jnp.sum(wt * wt, axis=1, keepdims=True)
    d = (zsq + wsq) + S2

    m = jnp.min(d, axis=0, keepdims=True)
    iota_col = jax.lax.broadcasted_iota(jnp.int32, (K, 1), 0).astype(
        jnp.float32)
    closest = jnp.min(jnp.where(d == m, iota_col, jnp.float32(K)),
                      axis=0)

    idx_ref[0, 0] = closest.astype(jnp.int32)

    c = m[0] * jnp.float32(1.0 / D)
    cl_ref[0, 0] = c
    el_ref[0, 0] = c
    loss_ref[0, 0] = c * jnp.float32(0.25) + c


def _tc_argmin(z3, W):
    B, D, N = z3.shape
    K = W.shape[1]
    f32 = jnp.float32
    return pl.pallas_call(
        _vq_argmin_kernel,
        grid=(B,),
        in_specs=[
            pl.BlockSpec((1, D, N), lambda b: (b, 0, 0)),
            pl.BlockSpec((D, K), lambda b: (0, 0)),
        ],
        out_specs=[
            pl.BlockSpec((1, 1, N), lambda b: (b, 0, 0)),
            pl.BlockSpec((1, 1, N), lambda b: (b, 0, 0)),
            pl.BlockSpec((1, 1, N), lambda b: (b, 0, 0)),
            pl.BlockSpec((1, 1, N), lambda b: (b, 0, 0)),
        ],
        out_shape=[
            jax.ShapeDtypeStruct((B, 1, N), jnp.int32),
            jax.ShapeDtypeStruct((B, 1, N), f32),
            jax.ShapeDtypeStruct((B, 1, N), f32),
            jax.ShapeDtypeStruct((B, 1, N), f32),
        ],
        compiler_params=pltpu.CompilerParams(
            dimension_semantics=("parallel",)),
    )(z3, W)


def _sc_gather(w_flat, idx_flat, B, D, N, K):
    info = plsc.get_sparse_core_info()
    NC, NS, L = info.num_cores, info.num_subcores, info.num_lanes
    NW = NC * NS
    WPB = NW // B
    CH = D // WPB
    U = 4

    mesh = plsc.VectorSubcoreMesh(core_axis_name="c", subcore_axis_name="s")

    @functools.partial(
        pl.kernel, mesh=mesh,
        out_type=jax.ShapeDtypeStruct((B * D * N,), jnp.float32),
        scratch_types=[
            pltpu.VMEM((CH * K,), jnp.float32),
            pltpu.VMEM((N,), jnp.int32),
            pltpu.VMEM((CH * N,), jnp.float32),
        ],
        compiler_params=pltpu.CompilerParams(needs_layout_passes=False),
    )
    def k(w_hbm, idx_hbm, out_hbm, wv, idxv, outv):
        wid = lax.axis_index("s") * NC + lax.axis_index("c")
        b = wid // WPB
        d0 = (wid % WPB) * CH
        pltpu.sync_copy(idx_hbm.at[pl.ds(b * N, N)], idxv)
        pltpu.sync_copy(w_hbm.at[pl.ds(d0 * K, CH * K)], wv)
        for dd in range(CH):
            base_w = dd * K
            base_o = dd * N

            def body(j, carry, base_w=base_w, base_o=base_o):
                for u in range(U):
                    off = (j * U + u) * L
                    iv = idxv[pl.ds(off, L)]
                    outv[pl.ds(base_o + off, L)] = plsc.load_gather(
                        wv, [iv + base_w])
                return carry

            lax.fori_loop(0, N // L // U, body, 0)
        pltpu.sync_copy(outv, out_hbm.at[pl.ds((b * D + d0) * N, CH * N)])

    return k(w_flat, idx_flat)


def kernel(z, W):
    B, D, H, Wd = z.shape
    N = H * Wd
    K = W.shape[1]
    z3 = z.reshape(B, D, N)
    closest, loss, cl, el = _tc_argmin(z3, W)
    qflat = _sc_gather(W.reshape(D * K), closest.reshape(B * N), B, D, N, K)
    shp = (B, H, Wd)
    return (qflat.reshape(z.shape), loss.reshape(shp), cl.reshape(shp),
            el.reshape(shp))


# speedup vs baseline: 1.4755x; 1.2007x over previous
import functools

import jax
import jax.numpy as jnp
from jax import lax
from jax.experimental import pallas as pl
from jax.experimental.pallas import tpu as pltpu
from jax.experimental.pallas import tpu_sc as plsc


def _vq_argmin_kernel(z_ref, w_ref, idx_ref, loss_ref, cl_ref, el_ref):
    zb = z_ref[0]
    W = w_ref[...]
    D, N = zb.shape
    K = W.shape[1]

    wt = W.T

    S2 = jax.lax.dot_general(wt * jnp.float32(-2.0), zb,
                             (((1,), (0,)), ((), ())),
                             preferred_element_type=jnp.float32)
    zsq = jnp.sum(zb * zb, axis=0, keepdims=True)
    wsq = jnp.sum(wt * wt, axis=1, keepdims=True)
    d = (zsq + wsq) + S2

    m = jnp.min(d, axis=0, keepdims=True)
    iota_col = jax.lax.broadcasted_iota(jnp.int32, (K, 1), 0).astype(
        jnp.float32)
    closest = jnp.min(jnp.where(d == m, iota_col, jnp.float32(K)),
                      axis=0)

    idx_ref[0, 0] = closest.astype(jnp.int32)

    c = m[0] * jnp.float32(1.0 / D)
    cl_ref[0, 0] = c
    el_ref[0, 0] = c
    loss_ref[0, 0] = c * jnp.float32(0.25) + c


def _tc_argmin(z3, W):
    B, D, N = z3.shape
    K = W.shape[1]
    f32 = jnp.float32
    return pl.pallas_call(
        _vq_argmin_kernel,
        grid=(B,),
        in_specs=[
            pl.BlockSpec((1, D, N), lambda b: (b, 0, 0)),
            pl.BlockSpec((D, K), lambda b: (0, 0)),
        ],
        out_specs=[
            pl.BlockSpec((1, 1, N), lambda b: (b, 0, 0)),
            pl.BlockSpec((1, 1, N), lambda b: (b, 0, 0)),
            pl.BlockSpec((1, 1, N), lambda b: (b, 0, 0)),
            pl.BlockSpec((1, 1, N), lambda b: (b, 0, 0)),
        ],
        out_shape=[
            jax.ShapeDtypeStruct((B, 1, N), jnp.int32),
            jax.ShapeDtypeStruct((B, 1, N), f32),
            jax.ShapeDtypeStruct((B, 1, N), f32),
            jax.ShapeDtypeStruct((B, 1, N), f32),
        ],
        compiler_params=pltpu.CompilerParams(
            dimension_semantics=("parallel",)),
    )(z3, W)


def _sc_gather(w_flat, idx_flat, B, D, N, K):
    info = plsc.get_sparse_core_info()
    NC, NS, L = info.num_cores, info.num_subcores, info.num_lanes
    NW = NC * NS
    WPB = NW // B
    CH = D // WPB
    U = 2

    mesh = plsc.VectorSubcoreMesh(core_axis_name="c", subcore_axis_name="s")

    @functools.partial(
        pl.kernel, mesh=mesh,
        out_type=jax.ShapeDtypeStruct((B * D * N,), jnp.float32),
        scratch_types=[
            pltpu.VMEM((CH * K,), jnp.float32),
            pltpu.VMEM((N,), jnp.int32),
            pltpu.VMEM((CH * N,), jnp.float32),
        ],
        compiler_params=pltpu.CompilerParams(needs_layout_passes=False),
    )
    def k(w_hbm, idx_hbm, out_hbm, wv, idxv, outv):
        wid = lax.axis_index("s") * NC + lax.axis_index("c")
        b = wid // WPB
        d0 = (wid % WPB) * CH
        pltpu.sync_copy(idx_hbm.at[pl.ds(b * N, N)], idxv)
        pltpu.sync_copy(w_hbm.at[pl.ds(d0 * K, CH * K)], wv)

        @plsc.parallel_loop(0, N // L, unroll=U)
        def _(j):
            off = j * L
            iv = idxv[pl.ds(off, L)]
            for dd in range(CH):
                outv[pl.ds(dd * N + off, L)] = plsc.load_gather(
                    wv, [iv + jnp.int32(dd * K)])

        pltpu.sync_copy(outv, out_hbm.at[pl.ds((b * D + d0) * N, CH * N)])

    return k(w_flat, idx_flat)


def kernel(z, W):
    B, D, H, Wd = z.shape
    N = H * Wd
    K = W.shape[1]
    z3 = z.reshape(B, D, N)
    closest, loss, cl, el = _tc_argmin(z3, W)
    qflat = _sc_gather(W.reshape(D * K), closest.reshape(B * N), B, D, N, K)
    shp = (B, H, Wd)
    return (qflat.reshape(z.shape), loss.reshape(shp), cl.reshape(shp),
            el.reshape(shp))


# speedup vs baseline: 1.4841x; 1.0058x over previous
import functools

import jax
import jax.numpy as jnp
from jax import lax
from jax.experimental import pallas as pl
from jax.experimental.pallas import tpu as pltpu
from jax.experimental.pallas import tpu_sc as plsc


def _vq_argmin_kernel(z_ref, w_ref, idx_ref, loss_ref, cl_ref, el_ref):
    zb = z_ref[0]
    W = w_ref[...]
    D, N = zb.shape
    K = W.shape[1]

    wt = W.T

    S2 = jax.lax.dot_general(wt * jnp.float32(-2.0), zb,
                             (((1,), (0,)), ((), ())),
                             preferred_element_type=jnp.float32)
    zsq = jnp.sum(zb * zb, axis=0, keepdims=True)
    wsq = jnp.sum(wt * wt, axis=1, keepdims=True)
    d = (zsq + wsq) + S2

    m = jnp.min(d, axis=0, keepdims=True)
    iota_col = jax.lax.broadcasted_iota(jnp.int32, (K, 1), 0).astype(
        jnp.float32)
    closest = jnp.min(jnp.where(d == m, iota_col, jnp.float32(K)),
                      axis=0)

    idx_ref[0, 0] = closest.astype(jnp.int32)

    c = m[0] * jnp.float32(1.0 / D)
    cl_ref[0, 0] = c
    el_ref[0, 0] = c
    loss_ref[0, 0] = c * jnp.float32(0.25) + c


def _tc_argmin(z3, W):
    B, D, N = z3.shape
    K = W.shape[1]
    f32 = jnp.float32
    return pl.pallas_call(
        _vq_argmin_kernel,
        grid=(B,),
        in_specs=[
            pl.BlockSpec((1, D, N), lambda b: (b, 0, 0)),
            pl.BlockSpec((D, K), lambda b: (0, 0)),
        ],
        out_specs=[
            pl.BlockSpec((1, 1, N), lambda b: (b, 0, 0)),
            pl.BlockSpec((1, 1, N), lambda b: (b, 0, 0)),
            pl.BlockSpec((1, 1, N), lambda b: (b, 0, 0)),
            pl.BlockSpec((1, 1, N), lambda b: (b, 0, 0)),
        ],
        out_shape=[
            jax.ShapeDtypeStruct((B, 1, N), jnp.int32),
            jax.ShapeDtypeStruct((B, 1, N), f32),
            jax.ShapeDtypeStruct((B, 1, N), f32),
            jax.ShapeDtypeStruct((B, 1, N), f32),
        ],
        compiler_params=pltpu.CompilerParams(
            dimension_semantics=("parallel",)),
    )(z3, W)


def _sc_gather(w_flat, idx_flat, B, D, N, K):
    info = plsc.get_sparse_core_info()
    NC, NS, L = info.num_cores, info.num_subcores, info.num_lanes
    NW = NC * NS
    WPB = NW // B
    CH = D // WPB
    U = 2

    mesh = plsc.VectorSubcoreMesh(core_axis_name="c", subcore_axis_name="s")

    @functools.partial(
        pl.kernel, mesh=mesh,
        out_type=jax.ShapeDtypeStruct((B * D * N,), jnp.float32),
        scratch_types=[
            pltpu.VMEM((CH * K,), jnp.float32),
            pltpu.VMEM((N,), jnp.int32),
            pltpu.VMEM((CH * N,), jnp.float32),
            pltpu.SemaphoreType.DMA,
            pltpu.SemaphoreType.DMA,
        ],
        compiler_params=pltpu.CompilerParams(needs_layout_passes=False),
    )
    def k(w_hbm, idx_hbm, out_hbm, wv, idxv, outv, sem1, sem2):
        wid = lax.axis_index("s") * NC + lax.axis_index("c")
        b = wid // WPB
        d0 = (wid % WPB) * CH
        cp1 = pltpu.async_copy(idx_hbm.at[pl.ds(b * N, N)], idxv, sem1)
        cp2 = pltpu.async_copy(w_hbm.at[pl.ds(d0 * K, CH * K)], wv, sem2)
        cp1.wait()
        cp2.wait()

        @plsc.parallel_loop(0, N // L, unroll=U)
        def _(j):
            off = j * L
            iv = idxv[pl.ds(off, L)]
            for dd in range(CH):
                outv[pl.ds(dd * N + off, L)] = plsc.load_gather(
                    wv, [iv + jnp.int32(dd * K)])

        pltpu.sync_copy(outv, out_hbm.at[pl.ds((b * D + d0) * N, CH * N)])

    return k(w_flat, idx_flat)


def kernel(z, W):
    B, D, H, Wd = z.shape
    N = H * Wd
    K = W.shape[1]
    z3 = z.reshape(B, D, N)
    closest, loss, cl, el = _tc_argmin(z3, W)
    qflat = _sc_gather(W.reshape(D * K), closest.reshape(B * N), B, D, N, K)
    shp = (B, H, Wd)
    return (qflat.reshape(z.shape), loss.reshape(shp), cl.reshape(shp),
            el.reshape(shp))
